# baseline (speedup 1.0000x reference)
import functools

import jax
import jax.numpy as jnp
from jax import lax
from jax.experimental import pallas as pl
from jax.experimental.pallas import tpu as pltpu
from jax.experimental.pallas import tpu_sc as plsc

N = 10000
E = 320000
D = 128
H = 128
C = 64

NP = 10240
NW = 32
GRP = 128
G = 80
EP = NW * G * GRP
ROWS_PER_TILE = NP // 16

_MESH = dict(core_axis_name="c", subcore_axis_name="s")


def _sc_degrees(srcp, dstp, ones_hbm, zeros_hbm):
    mesh = plsc.VectorSubcoreMesh(**_MESH)

    @functools.partial(
        pl.kernel,
        out_type=[jax.ShapeDtypeStruct((2, NP, 16), jnp.float32),
                  jax.ShapeDtypeStruct((2, NP, 16), jnp.float32)],
        mesh=mesh,
        scratch_types=[pltpu.VMEM((G, GRP), jnp.int32),
                       pltpu.VMEM((G, GRP), jnp.int32),
                       pltpu.VMEM((GRP, 16), jnp.float32),
                       pltpu.VMEM_SHARED((NP, 16), jnp.float32),
                       pltpu.VMEM_SHARED((NP, 16), jnp.float32)],
        compiler_params=pltpu.CompilerParams(use_tc_tiling_on_sc=False),
    )
    def k(src_hbm, dst_hbm, ones_h, zeros_h, dego_hbm, degi_hbm,
          sidx, didx, ones_v, acco, acci):
        c = lax.axis_index("c")
        s = lax.axis_index("s")
        wid = c * 16 + s
        r0 = s * ROWS_PER_TILE
        pltpu.sync_copy(zeros_h, acco.at[pl.ds(r0, ROWS_PER_TILE)])
        pltpu.sync_copy(zeros_h, acci.at[pl.ds(r0, ROWS_PER_TILE)])
        pltpu.sync_copy(ones_h, ones_v)
        pltpu.sync_copy(src_hbm.at[wid], sidx)
        pltpu.sync_copy(dst_hbm.at[wid], didx)
        plsc.subcore_barrier()

        @pl.loop(0, G)
        def _(g):
            pltpu.sync_copy(ones_v, acco.at[sidx.at[g]], add=True)
            pltpu.sync_copy(ones_v, acci.at[didx.at[g]], add=True)

        plsc.subcore_barrier()
        pltpu.sync_copy(acco.at[pl.ds(r0, ROWS_PER_TILE)],
                        dego_hbm.at[c].at[pl.ds(r0, ROWS_PER_TILE)])
        pltpu.sync_copy(acci.at[pl.ds(r0, ROWS_PER_TILE)],
                        degi_hbm.at[c].at[pl.ds(r0, ROWS_PER_TILE)])

    return k(srcp, dstp, ones_hbm, zeros_hbm)


def _sc_gather_scatter(hp, srcp, dstp, zeros_hbm, width):
    mesh = plsc.VectorSubcoreMesh(**_MESH)

    @functools.partial(
        pl.kernel,
        out_type=jax.ShapeDtypeStruct((2, NP, width), jnp.float32),
        mesh=mesh,
        scratch_types=[pltpu.VMEM((G // 2, GRP), jnp.int32),
                       pltpu.VMEM((G // 2, GRP), jnp.int32),
                       pltpu.VMEM((GRP, width), jnp.float32),
                       pltpu.VMEM((GRP, width), jnp.float32),
                       pltpu.VMEM_SHARED((NP, width), jnp.float32),
                       pltpu.SemaphoreType.DMA,
                       pltpu.SemaphoreType.DMA,
                       pltpu.SemaphoreType.DMA,
                       pltpu.SemaphoreType.DMA],
        compiler_params=pltpu.CompilerParams(use_tc_tiling_on_sc=False),
    )
    def k(hp_hbm, src_hbm, dst_hbm, zeros_h, out_hbm,
          sidx, didx, rows_a, rows_b, acc, sem_a, sem_b, ssem_a, ssem_b):
        c = lax.axis_index("c")
        s = lax.axis_index("s")
        wid = c * 16 + s
        r0 = s * ROWS_PER_TILE
        HG = G // 2
        pltpu.sync_copy(zeros_h, acc.at[pl.ds(r0, ROWS_PER_TILE)])
        plsc.subcore_barrier()

        def start_gather(g, rows, sem):
            pltpu.async_copy(hp_hbm.at[sidx.at[g]], rows, sem)

        def wait_gather(g, rows, sem):
            pltpu.make_async_copy(hp_hbm.at[sidx.at[g]], rows, sem).wait()

        def start_scatter(g, rows, sem):
            pltpu.async_copy(rows, acc.at[didx.at[g]], sem, add=True)

        def wait_scatter(g, rows, sem):
            pltpu.make_async_copy(rows, acc.at[didx.at[g]], sem).wait()

        for h in range(2):
            pltpu.sync_copy(src_hbm.at[wid].at[pl.ds(h * HG, HG)], sidx)
            pltpu.sync_copy(dst_hbm.at[wid].at[pl.ds(h * HG, HG)], didx)
            start_gather(0, rows_a, sem_a)
            start_gather(1, rows_b, sem_b)

            @pl.loop(0, HG // 2 - 1)
            def _(k2):
                g = 2 * k2
                wait_gather(g, rows_a, sem_a)
                start_scatter(g, rows_a, ssem_a)
                wait_gather(g + 1, rows_b, sem_b)
                start_scatter(g + 1, rows_b, ssem_b)
                wait_scatter(g, rows_a, ssem_a)
                start_gather(g + 2, rows_a, sem_a)
                wait_scatter(g + 1, rows_b, ssem_b)
                start_gather(g + 3, rows_b, sem_b)

            wait_gather(HG - 2, rows_a, sem_a)
            start_scatter(HG - 2, rows_a, ssem_a)
            wait_gather(HG - 1, rows_b, sem_b)
            start_scatter(HG - 1, rows_b, ssem_b)
            wait_scatter(HG - 2, rows_a, ssem_a)
            wait_scatter(HG - 1, rows_b, ssem_b)

        plsc.subcore_barrier()
        pltpu.sync_copy(acc.at[pl.ds(r0, ROWS_PER_TILE)],
                        out_hbm.at[c].at[pl.ds(r0, ROWS_PER_TILE)])

    return k(hp, srcp, dstp, zeros_hbm)


_RB = 512


def _dot(a, b):
    return jnp.dot(a, b, precision=lax.Precision.HIGHEST,
                   preferred_element_type=jnp.float32)


def _tc_linear1(xp, W1, b1r):
    def body(x_ref, w_ref, b_ref, o_ref):
        o_ref[...] = _dot(x_ref[...], w_ref[...]) + b_ref[...]

    return pl.pallas_call(
        body,
        grid=(NP // _RB,),
        in_specs=[pl.BlockSpec((_RB, D), lambda i: (i, 0)),
                  pl.BlockSpec((D, H), lambda i: (0, 0)),
                  pl.BlockSpec((1, H), lambda i: (0, 0))],
        out_specs=pl.BlockSpec((_RB, H), lambda i: (i, 0)),
        out_shape=jax.ShapeDtypeStruct((NP, H), jnp.float32),
    )(xp, W1, b1r)


def _rsqrt_deg(p0, p1):
    deg = p0[0][:, :1] + p1[0][:, :1]
    return lax.rsqrt(jnp.maximum(deg, 1.0))


def _tc_scale_a(h_raw, dego):
    def body(h_ref, d0_ref, d1_ref, o_ref):
        o_ref[...] = h_ref[...] * _rsqrt_deg(d0_ref, d1_ref)

    dspec = lambda core: pl.BlockSpec((1, _RB, 16), lambda i, c=core: (c, i, 0))
    return pl.pallas_call(
        body,
        grid=(NP // _RB,),
        in_specs=[pl.BlockSpec((_RB, H), lambda i: (i, 0)),
                  dspec(0), dspec(1)],
        out_specs=pl.BlockSpec((_RB, H), lambda i: (i, 0)),
        out_shape=jax.ShapeDtypeStruct((NP, H), jnp.float32),
    )(h_raw, dego, dego)


def _tc_layer2(p, dego, degi, W2, b2r):
    def body(p0_ref, p1_ref, di0, di1, do0, do1, w_ref, b_ref, o_ref):
        bcol = _rsqrt_deg(di0, di1)
        acol = _rsqrt_deg(do0, do1)
        h1 = jnp.maximum((p0_ref[0] + p1_ref[0]) * bcol, 0.0)
        o_ref[...] = (_dot(h1, w_ref[...]) + b_ref[...]) * acol

    pspec = lambda core: pl.BlockSpec((1, _RB, H), lambda i, c=core: (c, i, 0))
    dspec = lambda core: pl.BlockSpec((1, _RB, 16), lambda i, c=core: (c, i, 0))
    return pl.pallas_call(
        body,
        grid=(NP // _RB,),
        in_specs=[pspec(0), pspec(1), dspec(0), dspec(1), dspec(0), dspec(1),
                  pl.BlockSpec((H, C), lambda i: (0, 0)),
                  pl.BlockSpec((1, C), lambda i: (0, 0))],
        out_specs=pl.BlockSpec((_RB, C), lambda i: (i, 0)),
        out_shape=jax.ShapeDtypeStruct((NP, C), jnp.float32),
    )(p, p, degi, degi, dego, dego, W2, b2r)


def _tc_final(q, degi):
    def body(q0_ref, q1_ref, di0, di1, o_ref):
        o_ref[...] = (q0_ref[0] + q1_ref[0]) * _rsqrt_deg(di0, di1)

    qspec = lambda core: pl.BlockSpec((1, _RB, C), lambda i, c=core: (c, i, 0))
    dspec = lambda core: pl.BlockSpec((1, _RB, 16), lambda i, c=core: (c, i, 0))
    return pl.pallas_call(
        body,
        grid=(NP // _RB,),
        in_specs=[qspec(0), qspec(1), dspec(0), dspec(1)],
        out_specs=pl.BlockSpec((_RB, C), lambda i: (i, 0)),
        out_shape=jax.ShapeDtypeStruct((NP, C), jnp.float32),
    )(q, q, degi, degi)


def kernel(x, edge_index, W1, b1, W2, b2):
    src = edge_index[0].astype(jnp.int32)
    dst = edge_index[1].astype(jnp.int32)
    pad = (N + (jnp.arange(EP - E, dtype=jnp.int32) % (NP - N)))
    srcp = jnp.concatenate([src, pad]).reshape(NW, G, GRP)
    dstp = jnp.concatenate([dst, pad]).reshape(NW, G, GRP)
    xp = jnp.zeros((NP, D), jnp.float32).at[:N].set(x)

    ones16 = jnp.ones((GRP, 16), jnp.float32)
    zeros16 = jnp.zeros((ROWS_PER_TILE, 16), jnp.float32)
    zerosH = jnp.zeros((ROWS_PER_TILE, H), jnp.float32)
    zerosC = jnp.zeros((ROWS_PER_TILE, C), jnp.float32)

    dego, degi = _sc_degrees(srcp, dstp, ones16, zeros16)
    h_raw = _tc_linear1(xp, W1, b1.reshape(1, H))
    hp = _tc_scale_a(h_raw, dego)
    p = _sc_gather_scatter(hp, srcp, dstp, zerosH, H)
    h2p = _tc_layer2(p, dego, degi, W2, b2.reshape(1, C))
    q = _sc_gather_scatter(h2p, srcp, dstp, zerosC, C)
    out = _tc_final(q, degi)
    return out[:N]


# speedup vs baseline: 1.1534x; 1.1534x over previous
import functools

import jax
import jax.numpy as jnp
from jax import lax
from jax.experimental import pallas as pl
from jax.experimental.pallas import tpu as pltpu
from jax.experimental.pallas import tpu_sc as plsc

N = 10000
E = 320000
D = 128
H = 128
C = 64

NP = 10240
NW = 32
GRP = 128
G = 80
EP = NW * G * GRP
ROWS_PER_TILE = NP // 16

_MESH = dict(core_axis_name="c", subcore_axis_name="s")


def _sc_degrees(srcp, dstp, ones_hbm, zeros_hbm):
    mesh = plsc.VectorSubcoreMesh(**_MESH)

    @functools.partial(
        pl.kernel,
        out_type=[jax.ShapeDtypeStruct((2, NP, 16), jnp.float32),
                  jax.ShapeDtypeStruct((2, NP, 16), jnp.float32)],
        mesh=mesh,
        scratch_types=[pltpu.VMEM((G, GRP), jnp.int32),
                       pltpu.VMEM((G, GRP), jnp.int32),
                       pltpu.VMEM((GRP, 16), jnp.float32),
                       pltpu.VMEM_SHARED((NP, 16), jnp.float32),
                       pltpu.VMEM_SHARED((NP, 16), jnp.float32)],
        compiler_params=pltpu.CompilerParams(use_tc_tiling_on_sc=False),
    )
    def k(src_hbm, dst_hbm, ones_h, zeros_h, dego_hbm, degi_hbm,
          sidx, didx, ones_v, acco, acci):
        c = lax.axis_index("c")
        s = lax.axis_index("s")
        wid = c * 16 + s
        r0 = s * ROWS_PER_TILE
        pltpu.sync_copy(zeros_h, acco.at[pl.ds(r0, ROWS_PER_TILE)])
        pltpu.sync_copy(zeros_h, acci.at[pl.ds(r0, ROWS_PER_TILE)])
        pltpu.sync_copy(ones_h, ones_v)
        pltpu.sync_copy(src_hbm.at[wid], sidx)
        pltpu.sync_copy(dst_hbm.at[wid], didx)
        plsc.subcore_barrier()

        @pl.loop(0, G)
        def _(g):
            pltpu.sync_copy(ones_v, acco.at[sidx.at[g]], add=True)
            pltpu.sync_copy(ones_v, acci.at[didx.at[g]], add=True)

        plsc.subcore_barrier()
        pltpu.sync_copy(acco.at[pl.ds(r0, ROWS_PER_TILE)],
                        dego_hbm.at[c].at[pl.ds(r0, ROWS_PER_TILE)])
        pltpu.sync_copy(acci.at[pl.ds(r0, ROWS_PER_TILE)],
                        degi_hbm.at[c].at[pl.ds(r0, ROWS_PER_TILE)])

    return k(srcp, dstp, ones_hbm, zeros_hbm)


def _sc_gather_scatter(hp, srcp, dstp, zeros_hbm, width):
    mesh = plsc.VectorSubcoreMesh(**_MESH)

    @functools.partial(
        pl.kernel,
        out_type=jax.ShapeDtypeStruct((2, NP, width), jnp.float32),
        mesh=mesh,
        scratch_types=[pltpu.VMEM((G // 2, GRP), jnp.int32),
                       pltpu.VMEM((G // 2, GRP), jnp.int32),
                       pltpu.VMEM((GRP, width), jnp.float32),
                       pltpu.VMEM((GRP, width), jnp.float32),
                       pltpu.VMEM_SHARED((NP, width), jnp.float32),
                       pltpu.SemaphoreType.DMA,
                       pltpu.SemaphoreType.DMA],
        compiler_params=pltpu.CompilerParams(use_tc_tiling_on_sc=False),
    )
    def k(hp_hbm, src_hbm, dst_hbm, zeros_h, out_hbm,
          sidx, didx, rows_a, rows_b, acc, sem_a, sem_b):
        c = lax.axis_index("c")
        s = lax.axis_index("s")
        wid = c * 16 + s
        r0 = s * ROWS_PER_TILE
        HG = G // 2
        pltpu.sync_copy(zeros_h, acc.at[pl.ds(r0, ROWS_PER_TILE)])
        plsc.subcore_barrier()

        def start_gather(g, rows, sem):
            pltpu.async_copy(hp_hbm.at[sidx.at[g]], rows, sem)

        def wait_gather(g, rows, sem):
            pltpu.make_async_copy(hp_hbm.at[sidx.at[g]], rows, sem).wait()

        def scatter(g, rows):
            pltpu.sync_copy(rows, acc.at[didx.at[g]], add=True)

        for h in range(2):
            pltpu.sync_copy(src_hbm.at[wid].at[pl.ds(h * HG, HG)], sidx)
            pltpu.sync_copy(dst_hbm.at[wid].at[pl.ds(h * HG, HG)], didx)
            start_gather(0, rows_a, sem_a)
            start_gather(1, rows_b, sem_b)

            @pl.loop(0, HG // 2 - 1)
            def _(k2):
                g = 2 * k2
                wait_gather(g, rows_a, sem_a)
                scatter(g, rows_a)
                start_gather(g + 2, rows_a, sem_a)
                wait_gather(g + 1, rows_b, sem_b)
                scatter(g + 1, rows_b)
                start_gather(g + 3, rows_b, sem_b)

            wait_gather(HG - 2, rows_a, sem_a)
            scatter(HG - 2, rows_a)
            wait_gather(HG - 1, rows_b, sem_b)
            scatter(HG - 1, rows_b)

        plsc.subcore_barrier()
        pltpu.sync_copy(acc.at[pl.ds(r0, ROWS_PER_TILE)],
                        out_hbm.at[c].at[pl.ds(r0, ROWS_PER_TILE)])

    return k(hp, srcp, dstp, zeros_hbm)


_RB = 512


def _dot(a, b):
    return jnp.dot(a, b, precision=lax.Precision.HIGHEST,
                   preferred_element_type=jnp.float32)


def _rsqrt_deg(p0, p1):
    deg = p0[0][:, :1] + p1[0][:, :1]
    return lax.rsqrt(jnp.maximum(deg, 1.0))


def _tc_linear1(xp, W1, b1r, dego):
    def body(x_ref, w_ref, b_ref, d0_ref, d1_ref, o_ref):
        o_ref[...] = ((_dot(x_ref[...], w_ref[...]) + b_ref[...])
                      * _rsqrt_deg(d0_ref, d1_ref))

    dspec = lambda core: pl.BlockSpec((1, _RB, 16), lambda i, c=core: (c, i, 0))
    return pl.pallas_call(
        body,
        grid=(NP // _RB,),
        in_specs=[pl.BlockSpec((_RB, D), lambda i: (i, 0)),
                  pl.BlockSpec((D, H), lambda i: (0, 0)),
                  pl.BlockSpec((1, H), lambda i: (0, 0)),
                  dspec(0), dspec(1)],
        out_specs=pl.BlockSpec((_RB, H), lambda i: (i, 0)),
        out_shape=jax.ShapeDtypeStruct((NP, H), jnp.float32),
    )(xp, W1, b1r, dego, dego)


def _tc_layer2(p, dego, degi, W2, b2r):
    def body(p0_ref, p1_ref, di0, di1, do0, do1, w_ref, b_ref, o_ref):
        bcol = _rsqrt_deg(di0, di1)
        acol = _rsqrt_deg(do0, do1)
        h1 = jnp.maximum((p0_ref[0] + p1_ref[0]) * bcol, 0.0)
        o_ref[...] = (_dot(h1, w_ref[...]) + b_ref[...]) * acol

    pspec = lambda core: pl.BlockSpec((1, _RB, H), lambda i, c=core: (c, i, 0))
    dspec = lambda core: pl.BlockSpec((1, _RB, 16), lambda i, c=core: (c, i, 0))
    return pl.pallas_call(
        body,
        grid=(NP // _RB,),
        in_specs=[pspec(0), pspec(1), dspec(0), dspec(1), dspec(0), dspec(1),
                  pl.BlockSpec((H, C), lambda i: (0, 0)),
                  pl.BlockSpec((1, C), lambda i: (0, 0))],
        out_specs=pl.BlockSpec((_RB, C), lambda i: (i, 0)),
        out_shape=jax.ShapeDtypeStruct((NP, C), jnp.float32),
    )(p, p, degi, degi, dego, dego, W2, b2r)


_RF = 400


def _tc_final(q, degi):
    def body(q0_ref, q1_ref, di0, di1, o_ref):
        o_ref[...] = (q0_ref[0] + q1_ref[0]) * _rsqrt_deg(di0, di1)

    qspec = lambda core: pl.BlockSpec((1, _RF, C), lambda i, c=core: (c, i, 0))
    dspec = lambda core: pl.BlockSpec((1, _RF, 16), lambda i, c=core: (c, i, 0))
    return pl.pallas_call(
        body,
        grid=(N // _RF,),
        in_specs=[qspec(0), qspec(1), dspec(0), dspec(1)],
        out_specs=pl.BlockSpec((_RF, C), lambda i: (i, 0)),
        out_shape=jax.ShapeDtypeStruct((N, C), jnp.float32),
    )(q, q, degi, degi)


def kernel(x, edge_index, W1, b1, W2, b2):
    src = edge_index[0].astype(jnp.int32)
    dst = edge_index[1].astype(jnp.int32)
    pad = (N + (jnp.arange(EP - E, dtype=jnp.int32) % (NP - N)))
    srcp = jnp.concatenate([src, pad]).reshape(NW, G, GRP)
    dstp = jnp.concatenate([dst, pad]).reshape(NW, G, GRP)
    xp = jnp.zeros((NP, D), jnp.float32).at[:N].set(x)

    ones16 = jnp.ones((GRP, 16), jnp.float32)
    zeros16 = jnp.zeros((ROWS_PER_TILE, 16), jnp.float32)
    zerosH = jnp.zeros((ROWS_PER_TILE, H), jnp.float32)
    zerosC = jnp.zeros((ROWS_PER_TILE, C), jnp.float32)

    dego, degi = _sc_degrees(srcp, dstp, ones16, zeros16)
    hp = _tc_linear1(xp, W1, b1.reshape(1, H), dego)
    p = _sc_gather_scatter(hp, srcp, dstp, zerosH, H)
    h2p = _tc_layer2(p, dego, degi, W2, b2.reshape(1, C))
    q = _sc_gather_scatter(h2p, srcp, dstp, zerosC, C)
    return _tc_final(q, degi)


# speedup vs baseline: 1.2437x; 1.0783x over previous
import functools

import jax
import jax.numpy as jnp
from jax import lax
from jax.experimental import pallas as pl
from jax.experimental.pallas import tpu as pltpu
from jax.experimental.pallas import tpu_sc as plsc

N = 10000
E = 320000
D = 128
H = 128
C = 64

NP = 10240
NW = 32
GRP = 128
G = 80
EP = NW * G * GRP
ROWS_PER_TILE = NP // 16

_MESH = dict(core_axis_name="c", subcore_axis_name="s")


def _sc_degrees(srcp, dstp, ones_hbm, zeros_hbm):
    mesh = plsc.VectorSubcoreMesh(**_MESH)

    @functools.partial(
        pl.kernel,
        out_type=[jax.ShapeDtypeStruct((2, NP, 16), jnp.float32),
                  jax.ShapeDtypeStruct((2, NP, 16), jnp.float32)],
        mesh=mesh,
        scratch_types=[pltpu.VMEM((G, GRP), jnp.int32),
                       pltpu.VMEM((G, GRP), jnp.int32),
                       pltpu.VMEM((GRP, 16), jnp.float32),
                       pltpu.VMEM_SHARED((NP, 16), jnp.float32),
                       pltpu.VMEM_SHARED((NP, 16), jnp.float32)],
        compiler_params=pltpu.CompilerParams(use_tc_tiling_on_sc=False),
    )
    def k(src_hbm, dst_hbm, ones_h, zeros_h, dego_hbm, degi_hbm,
          sidx, didx, ones_v, acco, acci):
        c = lax.axis_index("c")
        s = lax.axis_index("s")
        wid = c * 16 + s
        r0 = s * ROWS_PER_TILE
        pltpu.sync_copy(zeros_h, acco.at[pl.ds(r0, ROWS_PER_TILE)])
        pltpu.sync_copy(zeros_h, acci.at[pl.ds(r0, ROWS_PER_TILE)])
        pltpu.sync_copy(ones_h, ones_v)
        pltpu.sync_copy(src_hbm.at[wid], sidx)
        pltpu.sync_copy(dst_hbm.at[wid], didx)
        plsc.subcore_barrier()

        @pl.loop(0, G)
        def _(g):
            pltpu.sync_copy(ones_v, acco.at[sidx.at[g]], add=True)
            pltpu.sync_copy(ones_v, acci.at[didx.at[g]], add=True)

        plsc.subcore_barrier()
        pltpu.sync_copy(acco.at[pl.ds(r0, ROWS_PER_TILE)],
                        dego_hbm.at[c].at[pl.ds(r0, ROWS_PER_TILE)])
        pltpu.sync_copy(acci.at[pl.ds(r0, ROWS_PER_TILE)],
                        degi_hbm.at[c].at[pl.ds(r0, ROWS_PER_TILE)])

    return k(srcp, dstp, ones_hbm, zeros_hbm)


def _sc_gather_scatter(hp, srcp, dstp, zeros_hbm, width):
    mesh = plsc.VectorSubcoreMesh(**_MESH)

    @functools.partial(
        pl.kernel,
        out_type=jax.ShapeDtypeStruct((2, NP, width), jnp.float32),
        mesh=mesh,
        scratch_types=[pltpu.VMEM((G // 2, GRP), jnp.int32),
                       pltpu.VMEM((G // 2, GRP), jnp.int32),
                       pltpu.VMEM((GRP, width), jnp.float32),
                       pltpu.VMEM((GRP, width), jnp.float32),
                       pltpu.VMEM_SHARED((NP, width), jnp.float32),
                       pltpu.SemaphoreType.DMA,
                       pltpu.SemaphoreType.DMA],
        compiler_params=pltpu.CompilerParams(use_tc_tiling_on_sc=False),
    )
    def k(hp_hbm, src_hbm, dst_hbm, zeros_h, out_hbm,
          sidx, didx, rows_a, rows_b, acc, sem_a, sem_b):
        c = lax.axis_index("c")
        s = lax.axis_index("s")
        wid = c * 16 + s
        r0 = s * ROWS_PER_TILE
        HG = G // 2
        pltpu.sync_copy(zeros_h, acc.at[pl.ds(r0, ROWS_PER_TILE)])
        plsc.subcore_barrier()

        def start_gather(g, rows, sem):
            pltpu.async_copy(hp_hbm.at[sidx.at[g]], rows, sem)

        def wait_gather(g, rows, sem):
            pltpu.make_async_copy(hp_hbm.at[sidx.at[g]], rows, sem).wait()

        def scatter(g, rows):
            pltpu.sync_copy(rows, acc.at[didx.at[g]], add=True)

        for h in range(2):
            pltpu.sync_copy(src_hbm.at[wid].at[pl.ds(h * HG, HG)], sidx)
            pltpu.sync_copy(dst_hbm.at[wid].at[pl.ds(h * HG, HG)], didx)
            start_gather(0, rows_a, sem_a)
            start_gather(1, rows_b, sem_b)

            @pl.loop(0, HG // 2 - 1)
            def _(k2):
                g = 2 * k2
                wait_gather(g, rows_a, sem_a)
                scatter(g, rows_a)
                start_gather(g + 2, rows_a, sem_a)
                wait_gather(g + 1, rows_b, sem_b)
                scatter(g + 1, rows_b)
                start_gather(g + 3, rows_b, sem_b)

            wait_gather(HG - 2, rows_a, sem_a)
            scatter(HG - 2, rows_a)
            wait_gather(HG - 1, rows_b, sem_b)
            scatter(HG - 1, rows_b)

        plsc.subcore_barrier()
        pltpu.sync_copy(acc.at[pl.ds(r0, ROWS_PER_TILE)],
                        out_hbm.at[c].at[pl.ds(r0, ROWS_PER_TILE)])

    return k(hp, srcp, dstp, zeros_hbm)


def _dot(a, b):
    return jnp.dot(a, b, precision=lax.Precision.HIGHEST,
                   preferred_element_type=jnp.float32)


def _rsqrt_deg(d_ref):
    deg = d_ref[0][:, :1] + d_ref[1][:, :1]
    return lax.rsqrt(jnp.maximum(deg, 1.0))


def _tc_linear1(xp, W1, b1r):
    def body(x_ref, w_ref, b_ref, o_ref):
        o_ref[...] = _dot(x_ref[...], w_ref[...]) + b_ref[...]

    return pl.pallas_call(
        body,
        out_shape=jax.ShapeDtypeStruct((NP, H), jnp.float32),
    )(xp, W1, b1r)


def _tc_scale_a(h_raw, dego):
    def body(h_ref, d_ref, o_ref):
        o_ref[...] = h_ref[...] * _rsqrt_deg(d_ref)

    return pl.pallas_call(
        body,
        out_shape=jax.ShapeDtypeStruct((NP, H), jnp.float32),
    )(h_raw, dego)


def _tc_layer2(p, dego, degi, W2, b2r):
    def body(p_ref, di_ref, do_ref, w_ref, b_ref, o_ref):
        bcol = _rsqrt_deg(di_ref)
        acol = _rsqrt_deg(do_ref)
        h1 = jnp.maximum((p_ref[0] + p_ref[1]) * bcol, 0.0)
        o_ref[...] = (_dot(h1, w_ref[...]) + b_ref[...]) * acol

    RB = 2048
    pspec = pl.BlockSpec((2, RB, H), lambda i: (0, i, 0))
    dspec = pl.BlockSpec((2, RB, 16), lambda i: (0, i, 0))
    return pl.pallas_call(
        body,
        grid=(NP // RB,),
        in_specs=[pspec, dspec, dspec,
                  pl.BlockSpec((H, C), lambda i: (0, 0)),
                  pl.BlockSpec((1, C), lambda i: (0, 0))],
        out_specs=pl.BlockSpec((RB, C), lambda i: (i, 0)),
        out_shape=jax.ShapeDtypeStruct((NP, C), jnp.float32),
    )(p, degi, dego, W2, b2r)


def _tc_final(q, degi):
    def body(q_ref, di_ref, o_ref):
        b = _rsqrt_deg(di_ref)
        o_ref[...] = ((q_ref[0] + q_ref[1]) * b)[:N]

    return pl.pallas_call(
        body,
        out_shape=jax.ShapeDtypeStruct((N, C), jnp.float32),
    )(q, degi)


def kernel(x, edge_index, W1, b1, W2, b2):
    src = edge_index[0].astype(jnp.int32)
    dst = edge_index[1].astype(jnp.int32)
    pad = (N + (jnp.arange(EP - E, dtype=jnp.int32) % (NP - N)))
    srcp = jnp.concatenate([src, pad]).reshape(NW, G, GRP)
    dstp = jnp.concatenate([dst, pad]).reshape(NW, G, GRP)
    xp = jnp.zeros((NP, D), jnp.float32).at[:N].set(x)

    ones16 = jnp.ones((GRP, 16), jnp.float32)
    zeros16 = jnp.zeros((ROWS_PER_TILE, 16), jnp.float32)
    zerosH = jnp.zeros((ROWS_PER_TILE, H), jnp.float32)
    zerosC = jnp.zeros((ROWS_PER_TILE, C), jnp.float32)

    dego, degi = _sc_degrees(srcp, dstp, ones16, zeros16)
    h_raw = _tc_linear1(xp, W1, b1.reshape(1, H))
    hp = _tc_scale_a(h_raw, dego)
    p = _sc_gather_scatter(hp, srcp, dstp, zerosH, H)
    h2p = _tc_layer2(p, dego, degi, W2, b2.reshape(1, C))
    q = _sc_gather_scatter(h2p, srcp, dstp, zerosC, C)
    return _tc_final(q, degi)


# speedup vs baseline: 1.2449x; 1.0010x over previous
import functools

import jax
import jax.numpy as jnp
from jax import lax
from jax.experimental import pallas as pl
from jax.experimental.pallas import tpu as pltpu
from jax.experimental.pallas import tpu_sc as plsc

N = 10000
E = 320000
D = 128
H = 128
C = 64

NP = 10240
NW = 32
GRP = 128
G = 80
EP = NW * G * GRP
ROWS_PER_TILE = NP // 16

_MESH = dict(core_axis_name="c", subcore_axis_name="s")


def _sc_degrees(srcp, dstp, ones_hbm, zeros_hbm):
    mesh = plsc.VectorSubcoreMesh(**_MESH)

    @functools.partial(
        pl.kernel,
        out_type=[jax.ShapeDtypeStruct((2, NP, 16), jnp.float32),
                  jax.ShapeDtypeStruct((2, NP, 16), jnp.float32)],
        mesh=mesh,
        scratch_types=[pltpu.VMEM((G, GRP), jnp.int32),
                       pltpu.VMEM((G, GRP), jnp.int32),
                       pltpu.VMEM((GRP, 16), jnp.float32),
                       pltpu.VMEM_SHARED((NP, 16), jnp.float32),
                       pltpu.VMEM_SHARED((NP, 16), jnp.float32)],
        compiler_params=pltpu.CompilerParams(use_tc_tiling_on_sc=False),
    )
    def k(src_hbm, dst_hbm, ones_h, zeros_h, dego_hbm, degi_hbm,
          sidx, didx, ones_v, acco, acci):
        c = lax.axis_index("c")
        s = lax.axis_index("s")
        wid = c * 16 + s
        r0 = s * ROWS_PER_TILE
        pltpu.sync_copy(zeros_h, acco.at[pl.ds(r0, ROWS_PER_TILE)])
        pltpu.sync_copy(zeros_h, acci.at[pl.ds(r0, ROWS_PER_TILE)])
        pltpu.sync_copy(ones_h, ones_v)
        pltpu.sync_copy(src_hbm.at[wid], sidx)
        pltpu.sync_copy(dst_hbm.at[wid], didx)
        plsc.subcore_barrier()

        @pl.loop(0, G)
        def _(g):
            pltpu.sync_copy(ones_v, acco.at[sidx.at[g]], add=True)
            pltpu.sync_copy(ones_v, acci.at[didx.at[g]], add=True)

        plsc.subcore_barrier()
        pltpu.sync_copy(acco.at[pl.ds(r0, ROWS_PER_TILE)],
                        dego_hbm.at[c].at[pl.ds(r0, ROWS_PER_TILE)])
        pltpu.sync_copy(acci.at[pl.ds(r0, ROWS_PER_TILE)],
                        degi_hbm.at[c].at[pl.ds(r0, ROWS_PER_TILE)])

    return k(srcp, dstp, ones_hbm, zeros_hbm)


def _sc_gather_scatter(hp, srcp, dstp, zeros_hbm, width):
    mesh = plsc.VectorSubcoreMesh(**_MESH)

    @functools.partial(
        pl.kernel,
        out_type=jax.ShapeDtypeStruct((2, NP, width), jnp.float32),
        mesh=mesh,
        scratch_types=[pltpu.VMEM((G // 2, GRP), jnp.int32),
                       pltpu.VMEM((G // 2, GRP), jnp.int32),
                       pltpu.VMEM((GRP, width), jnp.float32),
                       pltpu.VMEM((GRP, width), jnp.float32),
                       pltpu.VMEM_SHARED((NP, width), jnp.float32),
                       pltpu.SemaphoreType.DMA,
                       pltpu.SemaphoreType.DMA],
        compiler_params=pltpu.CompilerParams(
            use_tc_tiling_on_sc=(width == 128)),
    )
    def k(hp_hbm, src_hbm, dst_hbm, zeros_h, out_hbm,
          sidx, didx, rows_a, rows_b, acc, sem_a, sem_b):
        c = lax.axis_index("c")
        s = lax.axis_index("s")
        wid = c * 16 + s
        r0 = s * ROWS_PER_TILE
        HG = G // 2
        pltpu.sync_copy(zeros_h, acc.at[pl.ds(r0, ROWS_PER_TILE)])
        plsc.subcore_barrier()

        def start_gather(g, rows, sem):
            pltpu.async_copy(hp_hbm.at[sidx.at[g]], rows, sem)

        def wait_gather(g, rows, sem):
            pltpu.make_async_copy(hp_hbm.at[sidx.at[g]], rows, sem).wait()

        def scatter(g, rows):
            pltpu.sync_copy(rows, acc.at[didx.at[g]], add=True)

        for h in range(2):
            pltpu.sync_copy(src_hbm.at[wid].at[pl.ds(h * HG, HG)], sidx)
            pltpu.sync_copy(dst_hbm.at[wid].at[pl.ds(h * HG, HG)], didx)
            start_gather(0, rows_a, sem_a)
            start_gather(1, rows_b, sem_b)

            @pl.loop(0, HG // 2 - 1)
            def _(k2):
                g = 2 * k2
                wait_gather(g, rows_a, sem_a)
                scatter(g, rows_a)
                start_gather(g + 2, rows_a, sem_a)
                wait_gather(g + 1, rows_b, sem_b)
                scatter(g + 1, rows_b)
                start_gather(g + 3, rows_b, sem_b)

            wait_gather(HG - 2, rows_a, sem_a)
            scatter(HG - 2, rows_a)
            wait_gather(HG - 1, rows_b, sem_b)
            scatter(HG - 1, rows_b)

        plsc.subcore_barrier()
        pltpu.sync_copy(acc.at[pl.ds(r0, ROWS_PER_TILE)],
                        out_hbm.at[c].at[pl.ds(r0, ROWS_PER_TILE)])

    return k(hp, srcp, dstp, zeros_hbm)


def _dot(a, b):
    return jnp.dot(a, b, precision=lax.Precision.HIGHEST,
                   preferred_element_type=jnp.float32)


def _rsqrt_deg(d_ref):
    deg = d_ref[0][:, :1] + d_ref[1][:, :1]
    return lax.rsqrt(jnp.maximum(deg, 1.0))


def _tc_linear1(xp, W1, b1r):
    def body(x_ref, w_ref, b_ref, o_ref):
        o_ref[...] = _dot(x_ref[...], w_ref[...]) + b_ref[...]

    return pl.pallas_call(
        body,
        out_shape=jax.ShapeDtypeStruct((NP, H), jnp.float32),
    )(xp, W1, b1r)


def _tc_scale_a(h_raw, dego):
    def body(h_ref, d_ref, o_ref):
        o_ref[...] = h_ref[...] * _rsqrt_deg(d_ref)

    return pl.pallas_call(
        body,
        out_shape=jax.ShapeDtypeStruct((NP, H), jnp.float32),
    )(h_raw, dego)


def _tc_layer2(p, dego, degi, W2, b2r):
    def body(p_ref, di_ref, do_ref, w_ref, b_ref, o_ref):
        bcol = _rsqrt_deg(di_ref)
        acol = _rsqrt_deg(do_ref)
        h1 = jnp.maximum((p_ref[0] + p_ref[1]) * bcol, 0.0)
        o_ref[...] = (_dot(h1, w_ref[...]) + b_ref[...]) * acol

    RB = 2048
    pspec = pl.BlockSpec((2, RB, H), lambda i: (0, i, 0))
    dspec = pl.BlockSpec((2, RB, 16), lambda i: (0, i, 0))
    return pl.pallas_call(
        body,
        grid=(NP // RB,),
        in_specs=[pspec, dspec, dspec,
                  pl.BlockSpec((H, C), lambda i: (0, 0)),
                  pl.BlockSpec((1, C), lambda i: (0, 0))],
        out_specs=pl.BlockSpec((RB, C), lambda i: (i, 0)),
        out_shape=jax.ShapeDtypeStruct((NP, C), jnp.float32),
    )(p, degi, dego, W2, b2r)


def _tc_final(q, degi):
    def body(q_ref, di_ref, o_ref):
        b = _rsqrt_deg(di_ref)
        o_ref[...] = ((q_ref[0] + q_ref[1]) * b)[:N]

    return pl.pallas_call(
        body,
        out_shape=jax.ShapeDtypeStruct((N, C), jnp.float32),
    )(q, degi)


def kernel(x, edge_index, W1, b1, W2, b2):
    src = edge_index[0].astype(jnp.int32)
    dst = edge_index[1].astype(jnp.int32)
    pad = (N + (jnp.arange(EP - E, dtype=jnp.int32) % (NP - N)))
    srcp = jnp.concatenate([src, pad]).reshape(NW, G, GRP)
    dstp = jnp.concatenate([dst, pad]).reshape(NW, G, GRP)
    xp = jnp.zeros((NP, D), jnp.float32).at[:N].set(x)

    ones16 = jnp.ones((GRP, 16), jnp.float32)
    zeros16 = jnp.zeros((ROWS_PER_TILE, 16), jnp.float32)
    zerosH = jnp.zeros((ROWS_PER_TILE, H), jnp.float32)
    zerosC = jnp.zeros((ROWS_PER_TILE, C), jnp.float32)

    dego, degi = _sc_degrees(srcp, dstp, ones16, zeros16)
    h_raw = _tc_linear1(xp, W1, b1.reshape(1, H))
    hp = _tc_scale_a(h_raw, dego)
    p = _sc_gather_scatter(hp, srcp, dstp, zerosH, H)
    h2p = _tc_layer2(p, dego, degi, W2, b2.reshape(1, C))
    q = _sc_gather_scatter(h2p, srcp, dstp, zerosC, C)
    return _tc_final(q, degi)


# speedup vs baseline: 1.2719x; 1.0217x over previous
import functools

import jax
import jax.numpy as jnp
from jax import lax
from jax.experimental import pallas as pl
from jax.experimental.pallas import tpu as pltpu
from jax.experimental.pallas import tpu_sc as plsc

N = 10000
E = 320000
D = 128
H = 128
C = 64

NP = 10240
NW = 32
GRP = 128
G = 80
EP = NW * G * GRP
ROWS_PER_TILE = NP // 16

_MESH = dict(core_axis_name="c", subcore_axis_name="s")


def _sc_degree(idxp, ones_hbm, zeros_hbm):
    mesh = plsc.VectorSubcoreMesh(**_MESH)

    @functools.partial(
        pl.kernel,
        out_type=jax.ShapeDtypeStruct((2, NP, 16), jnp.float32),
        mesh=mesh,
        scratch_types=[pltpu.VMEM((G, GRP), jnp.int32),
                       pltpu.VMEM((GRP, 16), jnp.float32),
                       pltpu.VMEM_SHARED((NP, 16), jnp.float32)],
        compiler_params=pltpu.CompilerParams(use_tc_tiling_on_sc=False),
    )
    def k(idx_hbm, ones_h, zeros_h, deg_hbm, sidx, ones_v, acc):
        c = lax.axis_index("c")
        s = lax.axis_index("s")
        wid = c * 16 + s
        r0 = s * ROWS_PER_TILE
        pltpu.sync_copy(zeros_h, acc.at[pl.ds(r0, ROWS_PER_TILE)])
        pltpu.sync_copy(ones_h, ones_v)
        pltpu.sync_copy(idx_hbm.at[wid], sidx)
        plsc.subcore_barrier()

        @pl.loop(0, G)
        def _(g):
            pltpu.sync_copy(ones_v, acc.at[sidx.at[g]], add=True)

        plsc.subcore_barrier()
        pltpu.sync_copy(acc.at[pl.ds(r0, ROWS_PER_TILE)],
                        deg_hbm.at[c].at[pl.ds(r0, ROWS_PER_TILE)])

    return k(idxp, ones_hbm, zeros_hbm)


def _sc_gather_scatter(hp, srcp, dstp, zeros_hbm, width):
    mesh = plsc.VectorSubcoreMesh(**_MESH)

    @functools.partial(
        pl.kernel,
        out_type=jax.ShapeDtypeStruct((2, NP, width), jnp.float32),
        mesh=mesh,
        scratch_types=[pltpu.VMEM((G // 2, GRP), jnp.int32),
                       pltpu.VMEM((G // 2, GRP), jnp.int32),
                       pltpu.VMEM((GRP, width), jnp.float32),
                       pltpu.VMEM((GRP, width), jnp.float32),
                       pltpu.VMEM_SHARED((NP, width), jnp.float32),
                       pltpu.SemaphoreType.DMA,
                       pltpu.SemaphoreType.DMA],
        compiler_params=pltpu.CompilerParams(
            use_tc_tiling_on_sc=(width == 128)),
    )
    def k(hp_hbm, src_hbm, dst_hbm, zeros_h, out_hbm,
          sidx, didx, rows_a, rows_b, acc, sem_a, sem_b):
        c = lax.axis_index("c")
        s = lax.axis_index("s")
        wid = c * 16 + s
        r0 = s * ROWS_PER_TILE
        HG = G // 2
        pltpu.sync_copy(zeros_h, acc.at[pl.ds(r0, ROWS_PER_TILE)])
        plsc.subcore_barrier()

        def start_gather(g, rows, sem):
            pltpu.async_copy(hp_hbm.at[sidx.at[g]], rows, sem)

        def wait_gather(g, rows, sem):
            pltpu.make_async_copy(hp_hbm.at[sidx.at[g]], rows, sem).wait()

        def scatter(g, rows):
            pltpu.sync_copy(rows, acc.at[didx.at[g]], add=True)

        for h in range(2):
            pltpu.sync_copy(src_hbm.at[wid].at[pl.ds(h * HG, HG)], sidx)
            pltpu.sync_copy(dst_hbm.at[wid].at[pl.ds(h * HG, HG)], didx)
            start_gather(0, rows_a, sem_a)
            start_gather(1, rows_b, sem_b)

            @pl.loop(0, HG // 2 - 1)
            def _(k2):
                g = 2 * k2
                wait_gather(g, rows_a, sem_a)
                scatter(g, rows_a)
                start_gather(g + 2, rows_a, sem_a)
                wait_gather(g + 1, rows_b, sem_b)
                scatter(g + 1, rows_b)
                start_gather(g + 3, rows_b, sem_b)

            wait_gather(HG - 2, rows_a, sem_a)
            scatter(HG - 2, rows_a)
            wait_gather(HG - 1, rows_b, sem_b)
            scatter(HG - 1, rows_b)

        plsc.subcore_barrier()
        pltpu.sync_copy(acc.at[pl.ds(r0, ROWS_PER_TILE)],
                        out_hbm.at[c].at[pl.ds(r0, ROWS_PER_TILE)])

    return k(hp, srcp, dstp, zeros_hbm)


def _dot(a, b):
    return jnp.dot(a, b, precision=lax.Precision.HIGHEST,
                   preferred_element_type=jnp.float32)


def _rsqrt_deg(d_ref):
    deg = d_ref[0][:, :1] + d_ref[1][:, :1]
    return lax.rsqrt(jnp.maximum(deg, 1.0))


def _tc_linear1(xp, W1, b1r):
    def body(x_ref, w_ref, b_ref, o_ref):
        o_ref[...] = _dot(x_ref[...], w_ref[...]) + b_ref[...]

    return pl.pallas_call(
        body,
        out_shape=jax.ShapeDtypeStruct((NP, H), jnp.float32),
    )(xp, W1, b1r)


def _tc_scale_a(h_raw, dego):
    def body(h_ref, d_ref, o_ref):
        o_ref[...] = h_ref[...] * _rsqrt_deg(d_ref)

    return pl.pallas_call(
        body,
        out_shape=jax.ShapeDtypeStruct((NP, H), jnp.float32),
    )(h_raw, dego)


def _tc_layer2(p, dego, degi, W2, b2r):
    def body(p_ref, di_ref, do_ref, w_ref, b_ref, o_ref):
        bcol = _rsqrt_deg(di_ref)
        acol = _rsqrt_deg(do_ref)
        h1 = jnp.maximum((p_ref[0] + p_ref[1]) * bcol, 0.0)
        o_ref[...] = (_dot(h1, w_ref[...]) + b_ref[...]) * acol

    RB = 2048
    pspec = pl.BlockSpec((2, RB, H), lambda i: (0, i, 0))
    dspec = pl.BlockSpec((2, RB, 16), lambda i: (0, i, 0))
    return pl.pallas_call(
        body,
        grid=(NP // RB,),
        in_specs=[pspec, dspec, dspec,
                  pl.BlockSpec((H, C), lambda i: (0, 0)),
                  pl.BlockSpec((1, C), lambda i: (0, 0))],
        out_specs=pl.BlockSpec((RB, C), lambda i: (i, 0)),
        out_shape=jax.ShapeDtypeStruct((NP, C), jnp.float32),
    )(p, degi, dego, W2, b2r)


def _tc_final(q, degi):
    def body(q_ref, di_ref, o_ref):
        b = _rsqrt_deg(di_ref)
        o_ref[...] = ((q_ref[0] + q_ref[1]) * b)[:N]

    return pl.pallas_call(
        body,
        out_shape=jax.ShapeDtypeStruct((N, C), jnp.float32),
    )(q, degi)


def kernel(x, edge_index, W1, b1, W2, b2):
    src = edge_index[0].astype(jnp.int32)
    dst = edge_index[1].astype(jnp.int32)
    pad = (N + (jnp.arange(EP - E, dtype=jnp.int32) % (NP - N)))
    srcp = jnp.concatenate([src, pad]).reshape(NW, G, GRP)
    dstp = jnp.concatenate([dst, pad]).reshape(NW, G, GRP)
    xp = jnp.zeros((NP, D), jnp.float32).at[:N].set(x)

    ones16 = jnp.ones((GRP, 16), jnp.float32)
    zeros16 = jnp.zeros((ROWS_PER_TILE, 16), jnp.float32)
    zerosH = jnp.zeros((ROWS_PER_TILE, H), jnp.float32)
    zerosC = jnp.zeros((ROWS_PER_TILE, C), jnp.float32)

    dego = _sc_degree(srcp, ones16, zeros16)
    degi = _sc_degree(dstp, ones16, zeros16)
    h_raw = _tc_linear1(xp, W1, b1.reshape(1, H))
    hp = _tc_scale_a(h_raw, dego)
    p = _sc_gather_scatter(hp, srcp, dstp, zerosH, H)
    h2p = _tc_layer2(p, dego, degi, W2, b2.reshape(1, C))
    q = _sc_gather_scatter(h2p, srcp, dstp, zerosC, C)
    return _tc_final(q, degi)
